# baseline (speedup 1.0000x reference)
import functools

import jax
import jax.numpy as jnp
import numpy as np
from jax import lax
from jax.experimental import pallas as pl
from jax.experimental.pallas import tpu as pltpu
from jax.experimental.pallas import tpu_sc as plsc

N_NODES = 10000
N_EDGES = 320000
D_FEAT = 128
D_EDGE = 16
HIDDEN = 128

NC = 2
NS = 16
LANES = 16
NW = NC * NS
E_PER_W = N_EDGES // NW
BLK = 80
NBLK = E_PER_W // BLK
CH = 5
NCH = NBLK // CH
N_PAD = 10240
ROWS_PER_SUBCORE = N_PAD // NS

def _proj_body(h_ref, ws_ref, wt_ref, hs_ref, ht_ref):
    h = h_ref[...]
    hs_ref[...] = jnp.dot(h, ws_ref[...], preferred_element_type=jnp.float32)
    ht_ref[...] = jnp.dot(h, wt_ref[...], preferred_element_type=jnp.float32)


def _proj(h, ws, wt):
    return pl.pallas_call(
        _proj_body,
        out_shape=[
            jax.ShapeDtypeStruct((N_NODES, HIDDEN), jnp.float32),
            jax.ShapeDtypeStruct((N_NODES, HIDDEN), jnp.float32),
        ],
    )(h, ws, wt)


_EA_BLK = 8000


def _ea_body(a_ref, wa_ref, b_ref, o_ref):
    o_ref[...] = (
        jnp.dot(a_ref[...], wa_ref[...], preferred_element_type=jnp.float32)
        + b_ref[...]
    )


def _ea(edge_attr, wa, b_edge):
    return pl.pallas_call(
        _ea_body,
        grid=(N_EDGES // _EA_BLK,),
        in_specs=[
            pl.BlockSpec((_EA_BLK, D_EDGE), lambda i: (i, 0)),
            pl.BlockSpec((D_EDGE, HIDDEN), lambda i: (0, 0)),
            pl.BlockSpec((1, HIDDEN), lambda i: (0, 0)),
        ],
        out_specs=pl.BlockSpec((_EA_BLK, HIDDEN), lambda i: (i, 0)),
        out_shape=jax.ShapeDtypeStruct((N_EDGES, HIDDEN), jnp.float32),
    )(edge_attr, wa, b_edge.reshape(1, HIDDEN))


def _sc_edge_body(hs_hbm, ht_hbm, ea_hbm, row4_hbm, col4_hbm, out_hbm,
                  rowv, colv, hsb, htb, eab, agg, semG, semI):
    c = lax.axis_index("c")
    s = lax.axis_index("s")
    wid = c * NS + s
    base = wid * E_PER_W

    pltpu.sync_copy(row4_hbm.at[wid].at[0], rowv.at[0])
    pltpu.sync_copy(col4_hbm.at[wid].at[0], colv.at[0])

    @pl.loop(0, BLK)
    def _zero_rows(i):
        for j in range(0, HIDDEN, LANES):
            hsb[i, pl.ds(j, LANES)] = jnp.zeros((LANES,), jnp.float32)

    @pl.loop(0, ROWS_PER_SUBCORE, step=BLK)
    def _zero_copy(r):
        pltpu.sync_copy(hsb, agg.at[pl.ds(s * ROWS_PER_SUBCORE + r, BLK)])

    plsc.subcore_barrier()

    @pl.loop(0, NBLK)
    def _block(b):
        @pl.when((b % CH == 0) & (b < NBLK - CH))
        def _prefetch():
            cnext = b // CH + 1
            slot = cnext % 2
            pltpu.async_copy(row4_hbm.at[wid].at[cnext], rowv.at[slot], semI)
            pltpu.async_copy(col4_hbm.at[wid].at[cnext], colv.at[slot], semI)

        slot = (b // CH) % 2
        k = b % CH
        ri = rowv.at[slot].at[k]
        ci = colv.at[slot].at[k]

        pltpu.async_copy(hs_hbm.at[ri], hsb, semG)
        pltpu.async_copy(ht_hbm.at[ci], htb, semG)
        pltpu.async_copy(ea_hbm.at[pl.ds(base + b * BLK, BLK)], eab, semG)
        pltpu.make_async_copy(hs_hbm.at[ri], hsb, semG).wait()
        pltpu.make_async_copy(ht_hbm.at[ci], htb, semG).wait()
        pltpu.make_async_copy(
            ea_hbm.at[pl.ds(base + b * BLK, BLK)], eab, semG).wait()

        @pl.loop(0, BLK, unroll=2)
        def _edge(i):
            for j in range(0, HIDDEN, LANES):
                sl = pl.ds(j, LANES)
                hsb[i, sl] = jnp.maximum(
                    hsb[i, sl] + htb[i, sl] + eab[i, sl], 0.0)

        pltpu.sync_copy(hsb, agg.at[ri], add=True)

        @pl.when((b + 1) % CH == 0)
        def _wait_idx():
            cnext = (b + 1) // CH

            @pl.when(cnext < NCH)
            def _w():
                pltpu.make_async_copy(
                    row4_hbm.at[wid].at[cnext],
                    rowv.at[cnext % 2], semI).wait()
                pltpu.make_async_copy(
                    col4_hbm.at[wid].at[cnext],
                    colv.at[cnext % 2], semI).wait()

    plsc.subcore_barrier()
    r0 = s * ROWS_PER_SUBCORE
    pltpu.sync_copy(
        agg.at[pl.ds(r0, ROWS_PER_SUBCORE)],
        out_hbm.at[c].at[pl.ds(r0, ROWS_PER_SUBCORE)],
    )


def _sc_edge(hs, ht, ea, row, col):
    row4 = row.reshape(NW, NCH, CH, BLK)
    col4 = col.reshape(NW, NCH, CH, BLK)
    mesh = plsc.VectorSubcoreMesh(core_axis_name="c", subcore_axis_name="s")
    run = pl.kernel(
        _sc_edge_body,
        out_type=jax.ShapeDtypeStruct((NC, N_PAD, HIDDEN), jnp.float32),
        mesh=mesh,
        scratch_types=[
            pltpu.VMEM((2, CH, BLK), jnp.int32),
            pltpu.VMEM((2, CH, BLK), jnp.int32),
            pltpu.VMEM((BLK, HIDDEN), jnp.float32),
            pltpu.VMEM((BLK, HIDDEN), jnp.float32),
            pltpu.VMEM((BLK, HIDDEN), jnp.float32),
            pltpu.VMEM_SHARED((N_PAD, HIDDEN), jnp.float32),
            pltpu.SemaphoreType.DMA,
            pltpu.SemaphoreType.DMA,
        ],
    )
    return run(hs, ht, ea, row4, col4)


def _node_body(h_ref, aggp_ref, wh_ref, wg_ref, b_ref, o_ref):
    agg = aggp_ref[0, :N_NODES, :] + aggp_ref[1, :N_NODES, :]
    acc = jnp.dot(h_ref[...], wh_ref[...], preferred_element_type=jnp.float32)
    acc = acc + jnp.dot(agg, wg_ref[...], preferred_element_type=jnp.float32)
    o_ref[...] = jnp.maximum(acc + b_ref[...], 0.0)


def _node(h, aggp, wh, wg, b_node):
    return pl.pallas_call(
        _node_body,
        out_shape=jax.ShapeDtypeStruct((N_NODES, HIDDEN), jnp.float32),
    )(h, aggp, wh, wg, b_node.reshape(1, HIDDEN))


def kernel(h, edge_index, edge_attr, W_edge, b_edge, W_node, b_node):
    row = edge_index[0].astype(jnp.int32)
    col = edge_index[1].astype(jnp.int32)
    ws = W_edge[:, :D_FEAT].T
    wt = W_edge[:, D_FEAT:2 * D_FEAT].T
    wa = W_edge[:, 2 * D_FEAT:].T
    wh = W_node[:, :D_FEAT].T
    wg = W_node[:, D_FEAT:].T
    hs, ht = _proj(h, ws, wt)
    ea = _ea(edge_attr, wa, b_edge)
    aggp = _sc_edge(hs, ht, ea, row, col)
    return _node(h, aggp, wh, wg, b_node)


# speedup vs baseline: 1.2579x; 1.2579x over previous
import functools

import jax
import jax.numpy as jnp
import numpy as np
from jax import lax
from jax.experimental import pallas as pl
from jax.experimental.pallas import tpu as pltpu
from jax.experimental.pallas import tpu_sc as plsc

N_NODES = 10000
N_EDGES = 320000
D_FEAT = 128
D_EDGE = 16
HIDDEN = 128

NC = 2
NS = 16
LANES = 16
NW = NC * NS
E_PER_W = N_EDGES // NW
BLK = 40
NBLK = E_PER_W // BLK
CH = 10
NCH = NBLK // CH
N_PAD = 10240
ROWS_PER_SUBCORE = N_PAD // NS


def _proj_body(h_ref, ws_ref, wt_ref, wh_ref, b_ref, hs_ref, ht_ref, hb_ref):
    h = h_ref[...]
    hs_ref[...] = jnp.dot(h, ws_ref[...], preferred_element_type=jnp.float32)
    ht_ref[...] = jnp.dot(h, wt_ref[...], preferred_element_type=jnp.float32)
    hb_ref[...] = (
        jnp.dot(h, wh_ref[...], preferred_element_type=jnp.float32)
        + b_ref[...]
    )


def _proj(h, ws, wt, wh, b_node):
    return pl.pallas_call(
        _proj_body,
        out_shape=[
            jax.ShapeDtypeStruct((N_NODES, HIDDEN), jnp.float32),
            jax.ShapeDtypeStruct((N_NODES, HIDDEN), jnp.float32),
            jax.ShapeDtypeStruct((N_NODES, HIDDEN), jnp.float32),
        ],
    )(h, ws, wt, wh, b_node.reshape(1, HIDDEN))


_EA_BLK = 8000


def _ea_body(a_ref, wa_ref, b_ref, o_ref):
    o_ref[...] = (
        jnp.dot(a_ref[...], wa_ref[...], preferred_element_type=jnp.float32)
        + b_ref[...]
    )


def _ea(edge_attr, wa, b_edge):
    return pl.pallas_call(
        _ea_body,
        grid=(N_EDGES // _EA_BLK,),
        in_specs=[
            pl.BlockSpec((_EA_BLK, D_EDGE), lambda i: (i, 0)),
            pl.BlockSpec((D_EDGE, HIDDEN), lambda i: (0, 0)),
            pl.BlockSpec((1, HIDDEN), lambda i: (0, 0)),
        ],
        out_specs=pl.BlockSpec((_EA_BLK, HIDDEN), lambda i: (i, 0)),
        out_shape=jax.ShapeDtypeStruct((N_EDGES, HIDDEN), jnp.float32),
    )(edge_attr, wa, b_edge.reshape(1, HIDDEN))


def _sc_edge_body(hs_hbm, ht_hbm, ea_hbm, row4_hbm, col4_hbm, out_hbm,
                  rowv, colv, hsbA, htbA, eabA, hsbB, htbB, eabB, agg,
                  semA, semB, semSA, semSB, semI):
    c = lax.axis_index("c")
    s = lax.axis_index("s")
    wid = c * NS + s
    base = wid * E_PER_W

    pltpu.sync_copy(row4_hbm.at[wid].at[0], rowv.at[0])
    pltpu.sync_copy(col4_hbm.at[wid].at[0], colv.at[0])

    @pl.loop(0, BLK)
    def _zero_rows(i):
        for j in range(0, HIDDEN, LANES):
            hsbA[i, pl.ds(j, LANES)] = jnp.zeros((LANES,), jnp.float32)

    @pl.loop(0, ROWS_PER_SUBCORE, step=BLK)
    def _zero_copy(r):
        pltpu.sync_copy(hsbA, agg.at[pl.ds(s * ROWS_PER_SUBCORE + r, BLK)])

    plsc.subcore_barrier()

    bufs = ((hsbA, htbA, eabA, semA, semSA), (hsbB, htbB, eabB, semB, semSB))

    def idx_for(b):
        slot = (b // CH) % 3
        k = b % CH
        return rowv.at[slot].at[k], colv.at[slot].at[k]

    def drain_scatter(b, t):
        hsb = t[0]
        ri, _ = idx_for(b)
        pltpu.make_async_copy(hsb, agg.at[ri], t[4]).wait()

    def issue(b, t):
        hsb, htb, eab, sem, _ = t
        ri, ci = idx_for(b)
        pltpu.async_copy(hs_hbm.at[ri], hsb, sem)
        pltpu.async_copy(ht_hbm.at[ci], htb, sem)
        pltpu.async_copy(ea_hbm.at[pl.ds(base + b * BLK, BLK)], eab, sem)

    def consume(b, t):
        hsb, htb, eab, sem, semS = t
        ri, ci = idx_for(b)
        pltpu.make_async_copy(hs_hbm.at[ri], hsb, sem).wait()
        pltpu.make_async_copy(ht_hbm.at[ci], htb, sem).wait()
        pltpu.make_async_copy(
            ea_hbm.at[pl.ds(base + b * BLK, BLK)], eab, sem).wait()

        @pl.loop(0, BLK, unroll=2)
        def _edge(i):
            for j in range(0, HIDDEN, LANES):
                sl = pl.ds(j, LANES)
                hsb[i, sl] = jnp.maximum(
                    hsb[i, sl] + htb[i, sl] + eab[i, sl], 0.0)

        pltpu.async_copy(hsb, agg.at[ri], semS, add=True)

    issue(0, bufs[0])

    @pl.loop(0, NBLK - 1)
    def _block(b):
        @pl.when((b % CH == 0) & (b < NBLK - CH))
        def _prefetch():
            cnext = b // CH + 1
            slot = cnext % 3
            pltpu.async_copy(row4_hbm.at[wid].at[cnext], rowv.at[slot], semI)
            pltpu.async_copy(col4_hbm.at[wid].at[cnext], colv.at[slot], semI)

        @pl.when(b % 2 == 0)
        def _even():
            @pl.when(b >= 1)
            def _drain():
                drain_scatter(b - 1, bufs[1])

            issue(b + 1, bufs[1])
            consume(b, bufs[0])

        @pl.when(b % 2 == 1)
        def _odd():
            drain_scatter(b - 1, bufs[0])
            issue(b + 1, bufs[0])
            consume(b, bufs[1])

        @pl.when((b + 1) % CH == 0)
        def _wait_idx():
            cnext = (b + 1) // CH

            @pl.when(cnext < NCH)
            def _w():
                pltpu.make_async_copy(
                    row4_hbm.at[wid].at[cnext],
                    rowv.at[cnext % 3], semI).wait()
                pltpu.make_async_copy(
                    col4_hbm.at[wid].at[cnext],
                    colv.at[cnext % 3], semI).wait()

    consume(NBLK - 1, bufs[(NBLK - 1) % 2])
    drain_scatter(NBLK - 2, bufs[(NBLK - 2) % 2])
    drain_scatter(NBLK - 1, bufs[(NBLK - 1) % 2])

    plsc.subcore_barrier()
    r0 = s * ROWS_PER_SUBCORE
    pltpu.sync_copy(
        agg.at[pl.ds(r0, ROWS_PER_SUBCORE)],
        out_hbm.at[c].at[pl.ds(r0, ROWS_PER_SUBCORE)],
    )


def _sc_edge(hs, ht, ea, row, col):
    row4 = row.reshape(NW, NCH, CH, BLK)
    col4 = col.reshape(NW, NCH, CH, BLK)
    mesh = plsc.VectorSubcoreMesh(core_axis_name="c", subcore_axis_name="s")
    run = pl.kernel(
        _sc_edge_body,
        out_type=jax.ShapeDtypeStruct((NC, N_PAD, HIDDEN), jnp.float32),
        mesh=mesh,
        scratch_types=[
            pltpu.VMEM((3, CH, BLK), jnp.int32),
            pltpu.VMEM((3, CH, BLK), jnp.int32),
            pltpu.VMEM((BLK, HIDDEN), jnp.float32),
            pltpu.VMEM((BLK, HIDDEN), jnp.float32),
            pltpu.VMEM((BLK, HIDDEN), jnp.float32),
            pltpu.VMEM((BLK, HIDDEN), jnp.float32),
            pltpu.VMEM((BLK, HIDDEN), jnp.float32),
            pltpu.VMEM((BLK, HIDDEN), jnp.float32),
            pltpu.VMEM_SHARED((N_PAD, HIDDEN), jnp.float32),
            pltpu.SemaphoreType.DMA,
            pltpu.SemaphoreType.DMA,
            pltpu.SemaphoreType.DMA,
            pltpu.SemaphoreType.DMA,
            pltpu.SemaphoreType.DMA,
        ],
    )
    return run(hs, ht, ea, row4, col4)


def _node_body(hb_ref, aggp_ref, wg_ref, o_ref):
    agg = aggp_ref[0, :N_NODES, :] + aggp_ref[1, :N_NODES, :]
    acc = jnp.dot(agg, wg_ref[...], preferred_element_type=jnp.float32)
    o_ref[...] = jnp.maximum(hb_ref[...] + acc, 0.0)


def _node(hb, aggp, wg):
    return pl.pallas_call(
        _node_body,
        out_shape=jax.ShapeDtypeStruct((N_NODES, HIDDEN), jnp.float32),
    )(hb, aggp, wg)


def kernel(h, edge_index, edge_attr, W_edge, b_edge, W_node, b_node):
    row = edge_index[0].astype(jnp.int32)
    col = edge_index[1].astype(jnp.int32)
    ws = W_edge[:, :D_FEAT].T
    wt = W_edge[:, D_FEAT:2 * D_FEAT].T
    wa = W_edge[:, 2 * D_FEAT:].T
    wh = W_node[:, :D_FEAT].T
    wg = W_node[:, D_FEAT:].T
    hs, ht, hb = _proj(h, ws, wt, wh, b_node)
    ea = _ea(edge_attr, wa, b_edge)
    aggp = _sc_edge(hs, ht, ea, row, col)
    return _node(hb, aggp, wg)
